# Initial kernel scaffold; baseline (speedup 1.0000x reference)
import functools

import jax
import jax.numpy as jnp
from jax import lax
from jax.experimental import pallas as pl

_D = 768
_E = 64
_CAPF = 1.25


def _phase1_body(hs_ref, wt_ref, w_ref, e_ref, psum_ref, cnt_ref):
    x = hs_ref[...]
    wt = wt_ref[...]
    logits = jnp.dot(x, wt, preferred_element_type=jnp.float32)
    m = jnp.max(logits, axis=1, keepdims=True)
    ex = jnp.exp(logits - m)
    s = jnp.sum(ex, axis=1, keepdims=True)
    wmax = 1.0 / s
    c, e = logits.shape
    iota_e = lax.broadcasted_iota(jnp.int32, (c, e), 1)
    eidx = jnp.min(jnp.where(logits == m, iota_e, e), axis=1, keepdims=True)
    w_ref[...] = wmax
    e_ref[...] = eidx
    probs = ex * wmax
    psum_part = jnp.sum(probs, axis=0, keepdims=True)
    onehot = (iota_e == eidx).astype(jnp.float32)
    cnt_part = jnp.sum(onehot, axis=0, keepdims=True)

    @pl.when(pl.program_id(0) == 0)
    def _init():
        psum_ref[...] = jnp.zeros_like(psum_ref)
        cnt_ref[...] = jnp.zeros_like(cnt_ref)

    psum_ref[...] += jnp.broadcast_to(psum_part, psum_ref.shape)
    cnt_ref[...] += jnp.broadcast_to(cnt_part, cnt_ref.shape)


def _phase2_body(w_ref, e_ref, cnt_ref, psum_ref, wk_ref, aux_ref, *, cap):
    w = w_ref[...]
    ei = e_ref[...]
    n = w.shape[0]
    u = lax.bitcast_convert_type(w, jnp.int32)
    iota_e = lax.broadcasted_iota(jnp.int32, (1, _E), 1)
    onehot = ei == iota_e
    capf = jnp.float32(cap)

    def cnt_where(pred):
        return jnp.sum(jnp.where(pred & onehot, 1.0, 0.0), axis=0,
                       keepdims=True)

    def bs_body(_, carry):
        lo, hi = carry
        mid = lo + ((hi - lo) >> 1)
        ok = cnt_where(u >= mid) >= capf
        return jnp.where(ok, mid, lo), jnp.where(ok, hi, mid)

    lo0 = jnp.zeros((1, _E), jnp.int32)
    hi0 = jnp.full((1, _E), 0x3F800000, jnp.int32)
    t, _ = lax.fori_loop(0, 31, bs_body, (lo0, hi0))

    slots = capf - cnt_where(u > t)
    tok = lax.broadcasted_iota(jnp.int32, (n, 1), 0)
    tied = u == t

    def bs2_body(_, carry):
        lo, hi = carry
        mid = lo + ((hi - lo) >> 1)
        ok = cnt_where(tied & (tok < mid)) <= slots
        return jnp.where(ok, mid, lo), jnp.where(ok, hi, mid)

    lo0 = jnp.zeros((1, _E), jnp.int32)
    hi0 = jnp.full((1, _E), 65536, jnp.int32)
    istar, _ = lax.fori_loop(0, 17, bs2_body, (lo0, hi0))

    keep_mat = onehot & ((u > t) | (tied & (tok < istar)))
    keep = jnp.sum(jnp.where(keep_mat, 1.0, 0.0), axis=1, keepdims=True)
    wk_ref[...] = w * keep

    cnt_row = cnt_ref[0:1, :]
    psum_row = psum_ref[0:1, :]
    aux_ref[0, 0] = jnp.sum(cnt_row * psum_row) * (_E / (n * float(n)))


def kernel(hidden_states, W_gate):
    b, s, d = hidden_states.shape
    n = b * s
    e = W_gate.shape[0]
    cap = int(n * _CAPF / e)
    chunk = 2048
    grid = n // chunk
    hs2 = hidden_states.reshape(n, d)
    wt = W_gate.T

    w1, e1, psum, cnt = pl.pallas_call(
        _phase1_body,
        grid=(grid,),
        in_specs=[
            pl.BlockSpec((chunk, d), lambda i: (i, 0)),
            pl.BlockSpec((d, e), lambda i: (0, 0)),
        ],
        out_specs=[
            pl.BlockSpec((chunk, 1), lambda i: (i, 0)),
            pl.BlockSpec((chunk, 1), lambda i: (i, 0)),
            pl.BlockSpec((8, e), lambda i: (0, 0)),
            pl.BlockSpec((8, e), lambda i: (0, 0)),
        ],
        out_shape=[
            jax.ShapeDtypeStruct((n, 1), jnp.float32),
            jax.ShapeDtypeStruct((n, 1), jnp.int32),
            jax.ShapeDtypeStruct((8, e), jnp.float32),
            jax.ShapeDtypeStruct((8, e), jnp.float32),
        ],
    )(hs2, wt)

    wk, aux = pl.pallas_call(
        functools.partial(_phase2_body, cap=cap),
        out_shape=[
            jax.ShapeDtypeStruct((n, 1), jnp.float32),
            jax.ShapeDtypeStruct((1, 1), jnp.float32),
        ],
    )(w1, e1, cnt, psum)

    return (wk, e1, cnt[0], aux[0, 0])


# speedup vs baseline: 53.8484x; 53.8484x over previous
import functools

import jax
import jax.numpy as jnp
from jax import lax
from jax.experimental import pallas as pl

_D = 768
_E = 64
_CAPF = 1.25


def _phase1_body(hs_ref, wt_ref, w_ref, e_ref, psum_ref, cnt_ref):
    x = hs_ref[...]
    wt = wt_ref[...]
    logits = jnp.dot(x, wt, preferred_element_type=jnp.float32)
    m = jnp.max(logits, axis=1, keepdims=True)
    ex = jnp.exp(logits - m)
    s = jnp.sum(ex, axis=1, keepdims=True)
    wmax = 1.0 / s
    c, e = logits.shape
    iota_e = lax.broadcasted_iota(jnp.int32, (c, e), 1)
    eidx = jnp.min(jnp.where(logits == m, iota_e, e), axis=1, keepdims=True)
    w_ref[...] = wmax
    e_ref[...] = eidx
    probs = ex * wmax
    psum_part = jnp.sum(probs, axis=0, keepdims=True)
    onehot = (iota_e == eidx).astype(jnp.float32)
    cnt_part = jnp.sum(onehot, axis=0, keepdims=True)

    @pl.when(pl.program_id(0) == 0)
    def _init():
        psum_ref[...] = jnp.zeros_like(psum_ref)
        cnt_ref[...] = jnp.zeros_like(cnt_ref)

    psum_ref[...] += jnp.broadcast_to(psum_part, psum_ref.shape)
    cnt_ref[...] += jnp.broadcast_to(cnt_part, cnt_ref.shape)


_CH = 8192


def _phase2_body(w_ref, e_ref, cnt_ref, psum_ref, wk_ref, aux_ref, *, cap):
    n = w_ref.shape[1]
    nch = n // _CH
    capf = jnp.float32(cap)
    iota_es = lax.broadcasted_iota(jnp.int32, (_E, 1), 0)

    def chunk(k):
        wc = w_ref[0:1, pl.ds(k * _CH, _CH)]
        uc = lax.bitcast_convert_type(wc, jnp.int32)
        ec = e_ref[0:1, pl.ds(k * _CH, _CH)]
        oh = ec == iota_es
        return wc, uc, oh

    def cnt_where(pred_fn):
        def body(k, acc):
            _, uc, oh = chunk(k)
            m = jnp.where(oh & pred_fn(uc, k), 1.0, 0.0)
            return acc + jnp.sum(m, axis=1, keepdims=True)
        return lax.fori_loop(0, nch, body, jnp.zeros((_E, 1), jnp.float32))

    def bs_body(_, carry):
        lo, hi = carry
        mid = lo + ((hi - lo) >> 1)
        ok = cnt_where(lambda uc, k: uc >= mid) >= capf
        return jnp.where(ok, mid, lo), jnp.where(ok, hi, mid)

    lo0 = jnp.full((_E, 1), 0x3C000000, jnp.int32)
    hi0 = jnp.full((_E, 1), 0x3F800000, jnp.int32)
    t, _ = lax.fori_loop(0, 25, bs_body, (lo0, hi0))
    t = jnp.where(t == 0x3C000000, 0, t)

    slots = capf - cnt_where(lambda uc, k: uc > t)

    def tok_iota(k):
        return lax.broadcasted_iota(jnp.int32, (1, _CH), 1) + k * _CH

    def bs2_body(_, carry):
        lo, hi = carry
        mid = lo + ((hi - lo) >> 1)
        ok = cnt_where(lambda uc, k: (uc == t) & (tok_iota(k) < mid)) <= slots
        return jnp.where(ok, mid, lo), jnp.where(ok, hi, mid)

    lo2 = jnp.zeros((_E, 1), jnp.int32)
    hi2 = jnp.full((_E, 1), 65536, jnp.int32)
    istar, _ = lax.fori_loop(0, 17, bs2_body, (lo2, hi2))

    def keep_body(k, _):
        wc, uc, oh = chunk(k)
        km = oh & ((uc > t) | ((uc == t) & (tok_iota(k) < istar)))
        keep = jnp.sum(jnp.where(km, 1.0, 0.0), axis=0, keepdims=True)
        wk_ref[0:1, pl.ds(k * _CH, _CH)] = wc * keep
        return 0

    lax.fori_loop(0, nch, keep_body, 0)

    cnt_row = cnt_ref[0:1, :]
    psum_row = psum_ref[0:1, :]
    aux = jnp.sum(cnt_row * psum_row, axis=1, keepdims=True)
    aux_ref[...] = aux * (_E / (n * float(n)))


def kernel(hidden_states, W_gate):
    b, s, d = hidden_states.shape
    n = b * s
    e = W_gate.shape[0]
    cap = int(n * _CAPF / e)
    chunk = 2048
    grid = n // chunk
    hs2 = hidden_states.reshape(n, d)
    wt = W_gate.T

    w1, e1, psum, cnt = pl.pallas_call(
        _phase1_body,
        grid=(grid,),
        in_specs=[
            pl.BlockSpec((chunk, d), lambda i: (i, 0)),
            pl.BlockSpec((d, e), lambda i: (0, 0)),
        ],
        out_specs=[
            pl.BlockSpec((chunk, 1), lambda i: (i, 0)),
            pl.BlockSpec((chunk, 1), lambda i: (i, 0)),
            pl.BlockSpec((8, e), lambda i: (0, 0)),
            pl.BlockSpec((8, e), lambda i: (0, 0)),
        ],
        out_shape=[
            jax.ShapeDtypeStruct((n, 1), jnp.float32),
            jax.ShapeDtypeStruct((n, 1), jnp.int32),
            jax.ShapeDtypeStruct((8, e), jnp.float32),
            jax.ShapeDtypeStruct((8, e), jnp.float32),
        ],
    )(hs2, wt)

    wk, aux = pl.pallas_call(
        functools.partial(_phase2_body, cap=cap),
        out_shape=[
            jax.ShapeDtypeStruct((1, n), jnp.float32),
            jax.ShapeDtypeStruct((1, 1), jnp.float32),
        ],
    )(w1.reshape(1, n), e1.reshape(1, n), cnt, psum)

    return (wk.reshape(n, 1), e1, cnt[0], aux[0, 0])
